# initial kernel scaffold (unmeasured)
import functools

import jax
import jax.numpy as jnp
from jax import lax
import jax.experimental.pallas as pl
from jax.experimental.pallas import tpu as pltpu

N_DEV = 8
NBLK = 512
C16 = 32.0
S16 = C16 / 32767.0


def kernel(x, w_mat):
    m_per, k = x.shape
    _, n = w_mat.shape
    n_per = n // N_DEV
    halves = n_per // NBLK
    nblocks = n // NBLK

    def body(x_ref, w_ref, out_ref,
             xb, wbuf, sendq, recvq, gmax, gsrc,
             wsem, csend, crecv, gsend, grecv):
        my_i = lax.axis_index("i")

        bar = pltpu.get_barrier_semaphore()
        for p in range(N_DEV - 1):
            peer = (my_i + 1 + p) % N_DEV
            pltpu.semaphore_signal(
                bar, inc=1, device_id=(peer,),
                device_id_type=pltpu.DeviceIdType.MESH,
            )
        pltpu.semaphore_wait(bar, N_DEV - 1)

        xb[...] = x_ref[...].astype(jnp.bfloat16)
        gsrc[...] = jnp.zeros((1, 128), jnp.float32)

        def wcol(t, h):
            j = (my_i + 1 + t) % N_DEV
            return j * n_per + h * NBLK

        def w_dma(t, h, slot):
            return pltpu.make_async_copy(
                w_ref.at[:, pl.ds(wcol(t, h), NBLK)],
                wbuf.at[slot],
                wsem.at[slot],
            )

        w_dma(0, 0, 0).start()
        step = 0
        for t in range(N_DEV):
            for h in range(halves):
                slot = step % 2
                nxt = step + 1
                if nxt < nblocks:
                    w_dma(nxt // halves, nxt % halves, nxt % 2).start()
                w_dma(t, h, slot).wait()
                y = jnp.dot(
                    xb[...], wbuf[slot].astype(jnp.bfloat16),
                    preferred_element_type=jnp.float32,
                )
                y = jnp.maximum(y, 0.0)
                gsrc[...] = jnp.maximum(gsrc[...], jnp.max(y))
                sendq[t, :, h * NBLK:(h + 1) * NBLK] = (
                    jnp.round(y * (1.0 / S16)).astype(jnp.int16)
                )
                step += 1
            if t < N_DEV - 1:
                j = (my_i + 1 + t) % N_DEV
                pltpu.make_async_remote_copy(
                    src_ref=sendq.at[t],
                    dst_ref=recvq.at[my_i],
                    send_sem=csend.at[t],
                    recv_sem=crecv.at[my_i],
                    device_id=(j,),
                    device_id_type=pltpu.DeviceIdType.MESH,
                ).start()
            else:
                pltpu.make_async_copy(
                    sendq.at[t], recvq.at[my_i], crecv.at[my_i],
                ).start()

        pltpu.make_async_copy(
            gsrc, gmax.at[pl.ds(my_i, 1), :], grecv.at[my_i],
        ).start()
        for p in range(N_DEV - 1):
            peer = (my_i + 1 + p) % N_DEV
            pltpu.make_async_remote_copy(
                src_ref=gsrc,
                dst_ref=gmax.at[pl.ds(my_i, 1), :],
                send_sem=gsend.at[p],
                recv_sem=grecv.at[my_i],
                device_id=(peer,),
                device_id_type=pltpu.DeviceIdType.MESH,
            ).start()

        for s in range(N_DEV):
            pltpu.make_async_copy(recvq.at[s], recvq.at[s], crecv.at[s]).wait()
        for s in range(N_DEV):
            pltpu.make_async_copy(
                gmax.at[pl.ds(s, 1), :], gmax.at[pl.ds(s, 1), :], grecv.at[s]
            ).wait()
        for t in range(N_DEV - 1):
            pltpu.make_async_copy(sendq.at[t], sendq.at[t], csend.at[t]).wait()
        for p in range(N_DEV - 1):
            pltpu.make_async_copy(gsrc, gsrc, gsend.at[p]).wait()

        g = jnp.max(gmax[...])
        scale = g / 127.0
        inv = 127.0 / jnp.maximum(g, 1e-30)
        for s in range(N_DEV):
            yv = recvq[s].astype(jnp.float32) * S16
            q = jnp.clip(jnp.round(yv * inv), 0.0, 127.0)
            out_ref[pl.ds(s * m_per, m_per), :] = q * scale

        @functools.partial(
            pl.run_scoped, sem2=pltpu.SemaphoreType.REGULAR
        )
        def _(sem2):
            for p in range(N_DEV - 1):
                peer = (my_i + 1 + p) % N_DEV
                pltpu.semaphore_signal(
                    sem2, inc=1, device_id=(peer,),
                    device_id_type=pltpu.DeviceIdType.MESH,
                )
            pltpu.semaphore_wait(sem2, N_DEV - 1)

    return pl.pallas_call(
        body,
        out_shape=jax.ShapeDtypeStruct((N_DEV * m_per, n_per), jnp.float32),
        in_specs=[
            pl.BlockSpec(memory_space=pltpu.VMEM),
            pl.BlockSpec(memory_space=pltpu.ANY),
        ],
        out_specs=pl.BlockSpec(memory_space=pltpu.VMEM),
        scratch_shapes=[
            pltpu.VMEM((m_per, k), jnp.bfloat16),
            pltpu.VMEM((2, k, NBLK), jnp.float32),
            pltpu.VMEM((N_DEV, m_per, n_per), jnp.int16),
            pltpu.VMEM((N_DEV, m_per, n_per), jnp.int16),
            pltpu.VMEM((N_DEV, 128), jnp.float32),
            pltpu.VMEM((1, 128), jnp.float32),
            pltpu.SemaphoreType.DMA((2,)),
            pltpu.SemaphoreType.DMA((N_DEV - 1,)),
            pltpu.SemaphoreType.DMA((N_DEV,)),
            pltpu.SemaphoreType.DMA((N_DEV - 1,)),
            pltpu.SemaphoreType.DMA((N_DEV,)),
        ],
        compiler_params=pltpu.CompilerParams(collective_id=0),
    )(x, w_mat)


# baseline (device time: 110764 ns/iter reference)
import functools

import jax
import jax.numpy as jnp
from jax import lax
import jax.experimental.pallas as pl
from jax.experimental.pallas import tpu as pltpu

N_DEV = 8
NBLK = 512
C16 = 32.0
S16 = C16 / 32767.0


def kernel(x, w_mat):
    m_per, k = x.shape
    _, n = w_mat.shape
    n_per = n // N_DEV
    halves = n_per // NBLK
    nblocks = n // NBLK

    def body(x_ref, w_ref, out_ref,
             xb, wbuf, sendq, recvq, gmax, gsrc,
             wsem, csend, crecv, gsend, grecv):
        my_i = lax.axis_index("i")

        bar = pltpu.get_barrier_semaphore()
        for p in range(N_DEV - 1):
            peer = (my_i + 1 + p) % N_DEV
            pltpu.semaphore_signal(
                bar, inc=1, device_id=(peer,),
                device_id_type=pltpu.DeviceIdType.MESH,
            )
        pltpu.semaphore_wait(bar, N_DEV - 1)

        xb[...] = x_ref[...].astype(jnp.bfloat16)
        gsrc[...] = jnp.zeros((1, 128), jnp.float32)

        def wcol(t, h):
            j = (my_i + 1 + t) % N_DEV
            return j * n_per + h * NBLK

        def w_dma(t, h, slot):
            return pltpu.make_async_copy(
                w_ref.at[:, pl.ds(wcol(t, h), NBLK)],
                wbuf.at[slot],
                wsem.at[slot],
            )

        w_dma(0, 0, 0).start()
        step = 0
        for t in range(N_DEV):
            for h in range(halves):
                slot = step % 2
                nxt = step + 1
                if nxt < nblocks:
                    w_dma(nxt // halves, nxt % halves, nxt % 2).start()
                w_dma(t, h, slot).wait()
                y = jnp.dot(
                    xb[...], wbuf[slot].astype(jnp.bfloat16),
                    preferred_element_type=jnp.float32,
                )
                y = jnp.maximum(y, 0.0)
                gsrc[...] = jnp.maximum(gsrc[...], jnp.max(y))
                sendq[t, :, h * NBLK:(h + 1) * NBLK] = (
                    jnp.round(y * (1.0 / S16)).astype(jnp.int16)
                )
                step += 1
            if t < N_DEV - 1:
                j = (my_i + 1 + t) % N_DEV
                pltpu.make_async_remote_copy(
                    src_ref=sendq.at[t],
                    dst_ref=recvq.at[my_i],
                    send_sem=csend.at[t],
                    recv_sem=crecv.at[my_i],
                    device_id=(j,),
                    device_id_type=pltpu.DeviceIdType.MESH,
                ).start()
            else:
                pltpu.make_async_copy(
                    sendq.at[t], recvq.at[my_i], crecv.at[my_i],
                ).start()

        pltpu.make_async_copy(
            gsrc, gmax.at[pl.ds(my_i, 1), :], grecv.at[my_i],
        ).start()
        for p in range(N_DEV - 1):
            peer = (my_i + 1 + p) % N_DEV
            pltpu.make_async_remote_copy(
                src_ref=gsrc,
                dst_ref=gmax.at[pl.ds(my_i, 1), :],
                send_sem=gsend.at[p],
                recv_sem=grecv.at[my_i],
                device_id=(peer,),
                device_id_type=pltpu.DeviceIdType.MESH,
            ).start()

        for s in range(N_DEV):
            pltpu.make_async_copy(recvq.at[s], recvq.at[s], crecv.at[s]).wait()
        for s in range(N_DEV):
            pltpu.make_async_copy(
                gmax.at[pl.ds(s, 1), :], gmax.at[pl.ds(s, 1), :], grecv.at[s]
            ).wait()
        for t in range(N_DEV - 1):
            pltpu.make_async_copy(sendq.at[t], sendq.at[t], csend.at[t]).wait()
        for p in range(N_DEV - 1):
            pltpu.make_async_copy(gsrc, gsrc, gsend.at[p]).wait()

        g = jnp.max(gmax[...])
        scale = g / 127.0
        inv = 127.0 / jnp.maximum(g, 1e-30)
        for s in range(N_DEV):
            yv = recvq[s].astype(jnp.float32) * S16
            q = jnp.clip(jnp.round(yv * inv), 0.0, 127.0)
            out_ref[pl.ds(s * m_per, m_per), :] = q * scale

        @functools.partial(
            pl.run_scoped, sem2=pltpu.SemaphoreType.REGULAR
        )
        def _(sem2):
            for p in range(N_DEV - 1):
                peer = (my_i + 1 + p) % N_DEV
                pltpu.semaphore_signal(
                    sem2, inc=1, device_id=(peer,),
                    device_id_type=pltpu.DeviceIdType.MESH,
                )
            pltpu.semaphore_wait(sem2, N_DEV - 1)

    return pl.pallas_call(
        body,
        out_shape=jax.ShapeDtypeStruct((N_DEV * m_per, n_per), jnp.float32),
        in_specs=[
            pl.BlockSpec(memory_space=pltpu.VMEM),
            pl.BlockSpec(memory_space=pltpu.MemorySpace.HBM),
        ],
        out_specs=pl.BlockSpec(memory_space=pltpu.VMEM),
        scratch_shapes=[
            pltpu.VMEM((m_per, k), jnp.bfloat16),
            pltpu.VMEM((2, k, NBLK), jnp.float32),
            pltpu.VMEM((N_DEV, m_per, n_per), jnp.int16),
            pltpu.VMEM((N_DEV, m_per, n_per), jnp.int16),
            pltpu.VMEM((N_DEV, 128), jnp.float32),
            pltpu.VMEM((1, 128), jnp.float32),
            pltpu.SemaphoreType.DMA((2,)),
            pltpu.SemaphoreType.DMA((N_DEV - 1,)),
            pltpu.SemaphoreType.DMA((N_DEV,)),
            pltpu.SemaphoreType.DMA((N_DEV - 1,)),
            pltpu.SemaphoreType.DMA((N_DEV,)),
        ],
        compiler_params=pltpu.CompilerParams(
            collective_id=0,
            vmem_limit_bytes=100 * 1024 * 1024,
        ),
    )(x, w_mat)


# device time: 68831 ns/iter; 1.6092x vs baseline; 1.6092x over previous
import functools
import os

import jax
import jax.numpy as jnp
from jax import lax
import jax.experimental.pallas as pl
from jax.experimental.pallas import tpu as pltpu

N_DEV = 8
NBLK = 512
VARIANT = os.environ.get("KVARIANT", "full")
C16 = 32.0
S16 = C16 / 32767.0


def kernel(x, w_mat):
    m_per, k = x.shape
    _, n = w_mat.shape
    n_per = n // N_DEV
    halves = n_per // NBLK
    nblocks = n // NBLK

    def body(x_ref, w_ref, out_ref,
             xb, wbuf, sendq, recvq, gmax, gsrc,
             wsem, csend, crecv, gsend, grecv):
        my_i = lax.axis_index("i")

        bar = pltpu.get_barrier_semaphore()
        for p in range(N_DEV - 1):
            peer = (my_i + 1 + p) % N_DEV
            pltpu.semaphore_signal(
                bar, inc=1, device_id=(peer,),
                device_id_type=pltpu.DeviceIdType.MESH,
            )
        pltpu.semaphore_wait(bar, N_DEV - 1)

        xb[...] = x_ref[...].astype(jnp.bfloat16)
        gsrc[...] = jnp.zeros((1, 128), jnp.float32)

        def wcol(t, h):
            j = (my_i + 1 + t) % N_DEV
            return j * n_per + h * NBLK

        def w_dma(t, h, slot):
            return pltpu.make_async_copy(
                w_ref.at[:, pl.ds(wcol(t, h), NBLK)],
                wbuf.at[slot],
                wsem.at[slot],
            )

        w_dma(0, 0, 0).start()
        step = 0
        for t in range(N_DEV):
            for h in range(halves):
                slot = step % 2
                nxt = step + 1
                if nxt < nblocks:
                    w_dma(nxt // halves, nxt % halves, nxt % 2).start()
                w_dma(t, h, slot).wait()
                if VARIANT != "dmaonly":
                    y = jnp.dot(
                        xb[...], wbuf[slot].astype(jnp.bfloat16),
                        preferred_element_type=jnp.float32,
                    )
                    y = jnp.maximum(y, 0.0)
                    gsrc[...] = jnp.maximum(gsrc[...], jnp.max(y))
                    sendq[t, :, h * NBLK:(h + 1) * NBLK] = (
                        jnp.round(y * (1.0 / S16)).astype(jnp.int16)
                    )
                step += 1
            if VARIANT != "full":
                pass
            elif t < N_DEV - 1:
                j = (my_i + 1 + t) % N_DEV
                pltpu.make_async_remote_copy(
                    src_ref=sendq.at[t],
                    dst_ref=recvq.at[my_i],
                    send_sem=csend.at[t],
                    recv_sem=crecv.at[my_i],
                    device_id=(j,),
                    device_id_type=pltpu.DeviceIdType.MESH,
                ).start()
            else:
                pltpu.make_async_copy(
                    sendq.at[t], recvq.at[my_i], crecv.at[my_i],
                ).start()

        if VARIANT == "full":
            pltpu.make_async_copy(
                gsrc, gmax.at[pl.ds(my_i, 1), :], grecv.at[my_i],
            ).start()
            for p in range(N_DEV - 1):
                peer = (my_i + 1 + p) % N_DEV
                pltpu.make_async_remote_copy(
                    src_ref=gsrc,
                    dst_ref=gmax.at[pl.ds(my_i, 1), :],
                    send_sem=gsend.at[p],
                    recv_sem=grecv.at[my_i],
                    device_id=(peer,),
                    device_id_type=pltpu.DeviceIdType.MESH,
                ).start()

            for s in range(N_DEV):
                pltpu.make_async_copy(
                    recvq.at[s], recvq.at[s], crecv.at[s]
                ).wait()
            for s in range(N_DEV):
                pltpu.make_async_copy(
                    gmax.at[pl.ds(s, 1), :], gmax.at[pl.ds(s, 1), :],
                    grecv.at[s],
                ).wait()
            for t in range(N_DEV - 1):
                pltpu.make_async_copy(
                    sendq.at[t], sendq.at[t], csend.at[t]
                ).wait()
            for p in range(N_DEV - 1):
                pltpu.make_async_copy(gsrc, gsrc, gsend.at[p]).wait()

        if VARIANT == "dmaonly":
            for s in range(N_DEV):
                out_ref[pl.ds(s * m_per, m_per), :] = jnp.zeros(
                    (m_per, n_per), jnp.float32
                )
        else:
            srcq = recvq if VARIANT == "full" else sendq
            g = jnp.max(gmax[...]) if VARIANT == "full" else jnp.max(gsrc[...])
            scale = g / 127.0
            inv = 127.0 / jnp.maximum(g, 1e-30)
            for s in range(N_DEV):
                yv = srcq[s].astype(jnp.float32) * S16
                q = jnp.clip(jnp.round(yv * inv), 0.0, 127.0)
                out_ref[pl.ds(s * m_per, m_per), :] = q * scale

        @functools.partial(
            pl.run_scoped, sem2=pltpu.SemaphoreType.REGULAR
        )
        def _(sem2):
            for p in range(N_DEV - 1):
                peer = (my_i + 1 + p) % N_DEV
                pltpu.semaphore_signal(
                    sem2, inc=1, device_id=(peer,),
                    device_id_type=pltpu.DeviceIdType.MESH,
                )
            pltpu.semaphore_wait(sem2, N_DEV - 1)

    return pl.pallas_call(
        body,
        out_shape=jax.ShapeDtypeStruct((N_DEV * m_per, n_per), jnp.float32),
        in_specs=[
            pl.BlockSpec(memory_space=pltpu.VMEM),
            pl.BlockSpec(memory_space=pltpu.MemorySpace.HBM),
        ],
        out_specs=pl.BlockSpec(memory_space=pltpu.VMEM),
        scratch_shapes=[
            pltpu.VMEM((m_per, k), jnp.bfloat16),
            pltpu.VMEM((2, k, NBLK), jnp.float32),
            pltpu.VMEM((N_DEV, m_per, n_per), jnp.int16),
            pltpu.VMEM((N_DEV, m_per, n_per), jnp.int16),
            pltpu.VMEM((N_DEV, 128), jnp.float32),
            pltpu.VMEM((1, 128), jnp.float32),
            pltpu.SemaphoreType.DMA((2,)),
            pltpu.SemaphoreType.DMA((N_DEV - 1,)),
            pltpu.SemaphoreType.DMA((N_DEV,)),
            pltpu.SemaphoreType.DMA((N_DEV - 1,)),
            pltpu.SemaphoreType.DMA((N_DEV,)),
        ],
        compiler_params=pltpu.CompilerParams(
            collective_id=0,
            vmem_limit_bytes=100 * 1024 * 1024,
        ),
    )(x, w_mat)


# device time: 68707 ns/iter; 1.6121x vs baseline; 1.0018x over previous
import functools
import os

import jax
import jax.numpy as jnp
from jax import lax
import jax.experimental.pallas as pl
from jax.experimental.pallas import tpu as pltpu

N_DEV = 8
NBLK = 512
VARIANT = os.environ.get("KVARIANT", "full")
WSPLIT = int(os.environ.get("KWSPLIT", "4"))
C16 = 32.0
S16 = C16 / 32767.0


def kernel(x, w_mat):
    m_per, k = x.shape
    _, n = w_mat.shape
    n_per = n // N_DEV
    halves = n_per // NBLK
    nblocks = n // NBLK

    def body(x_ref, w_ref, out_ref,
             xb, wbuf, sendq, recvq, gmax, gsrc,
             wsem, csend, crecv, gsend, grecv):
        my_i = lax.axis_index("i")

        bar = pltpu.get_barrier_semaphore()
        for p in range(N_DEV - 1):
            peer = (my_i + 1 + p) % N_DEV
            pltpu.semaphore_signal(
                bar, inc=1, device_id=(peer,),
                device_id_type=pltpu.DeviceIdType.MESH,
            )
        pltpu.semaphore_wait(bar, N_DEV - 1)

        xb[...] = x_ref[...].astype(jnp.bfloat16)
        gsrc[...] = jnp.zeros((1, 128), jnp.float32)

        def wcol(t, h):
            j = (my_i + 1 + t) % N_DEV
            return j * n_per + h * NBLK

        def w_dmas(t, h, slot):
            rows = k // WSPLIT
            return [
                pltpu.make_async_copy(
                    w_ref.at[pl.ds(r * rows, rows), pl.ds(wcol(t, h), NBLK)],
                    wbuf.at[slot, pl.ds(r * rows, rows), :],
                    wsem.at[slot, r],
                )
                for r in range(WSPLIT)
            ]

        def w_start(t, h, slot):
            for d in w_dmas(t, h, slot):
                d.start()

        def w_wait(t, h, slot):
            for d in w_dmas(t, h, slot):
                d.wait()

        w_start(0, 0, 0)
        step = 0
        for t in range(N_DEV):
            for h in range(halves):
                slot = step % 2
                nxt = step + 1
                if nxt < nblocks:
                    w_start(nxt // halves, nxt % halves, nxt % 2)
                w_wait(t, h, slot)
                if VARIANT != "dmaonly":
                    y = jnp.dot(
                        xb[...], wbuf[slot].astype(jnp.bfloat16),
                        preferred_element_type=jnp.float32,
                    )
                    y = jnp.maximum(y, 0.0)
                    gsrc[...] = jnp.maximum(gsrc[...], jnp.max(y))
                    sendq[t, :, h * NBLK:(h + 1) * NBLK] = (
                        jnp.round(y * (1.0 / S16)).astype(jnp.int16)
                    )
                step += 1
            if VARIANT != "full":
                pass
            elif t < N_DEV - 1:
                j = (my_i + 1 + t) % N_DEV
                pltpu.make_async_remote_copy(
                    src_ref=sendq.at[t],
                    dst_ref=recvq.at[my_i],
                    send_sem=csend.at[t],
                    recv_sem=crecv.at[my_i],
                    device_id=(j,),
                    device_id_type=pltpu.DeviceIdType.MESH,
                ).start()
            else:
                pltpu.make_async_copy(
                    sendq.at[t], recvq.at[my_i], crecv.at[my_i],
                ).start()

        if VARIANT == "full":
            pltpu.make_async_copy(
                gsrc, gmax.at[pl.ds(my_i, 1), :], grecv.at[my_i],
            ).start()
            for p in range(N_DEV - 1):
                peer = (my_i + 1 + p) % N_DEV
                pltpu.make_async_remote_copy(
                    src_ref=gsrc,
                    dst_ref=gmax.at[pl.ds(my_i, 1), :],
                    send_sem=gsend.at[p],
                    recv_sem=grecv.at[my_i],
                    device_id=(peer,),
                    device_id_type=pltpu.DeviceIdType.MESH,
                ).start()

            for s in range(N_DEV):
                pltpu.make_async_copy(
                    recvq.at[s], recvq.at[s], crecv.at[s]
                ).wait()
            for s in range(N_DEV):
                pltpu.make_async_copy(
                    gmax.at[pl.ds(s, 1), :], gmax.at[pl.ds(s, 1), :],
                    grecv.at[s],
                ).wait()
            for t in range(N_DEV - 1):
                pltpu.make_async_copy(
                    sendq.at[t], sendq.at[t], csend.at[t]
                ).wait()
            for p in range(N_DEV - 1):
                pltpu.make_async_copy(gsrc, gsrc, gsend.at[p]).wait()

        if VARIANT == "dmaonly":
            for s in range(N_DEV):
                out_ref[pl.ds(s * m_per, m_per), :] = jnp.zeros(
                    (m_per, n_per), jnp.float32
                )
        else:
            srcq = recvq if VARIANT == "full" else sendq
            g = jnp.max(gmax[...]) if VARIANT == "full" else jnp.max(gsrc[...])
            scale = g / 127.0
            inv = 127.0 / jnp.maximum(g, 1e-30)
            for s in range(N_DEV):
                yv = srcq[s].astype(jnp.float32) * S16
                q = jnp.clip(jnp.round(yv * inv), 0.0, 127.0)
                out_ref[pl.ds(s * m_per, m_per), :] = q * scale

        @functools.partial(
            pl.run_scoped, sem2=pltpu.SemaphoreType.REGULAR
        )
        def _(sem2):
            for p in range(N_DEV - 1):
                peer = (my_i + 1 + p) % N_DEV
                pltpu.semaphore_signal(
                    sem2, inc=1, device_id=(peer,),
                    device_id_type=pltpu.DeviceIdType.MESH,
                )
            pltpu.semaphore_wait(sem2, N_DEV - 1)

    return pl.pallas_call(
        body,
        out_shape=jax.ShapeDtypeStruct((N_DEV * m_per, n_per), jnp.float32),
        in_specs=[
            pl.BlockSpec(memory_space=pltpu.VMEM),
            pl.BlockSpec(memory_space=pltpu.MemorySpace.HBM),
        ],
        out_specs=pl.BlockSpec(memory_space=pltpu.VMEM),
        scratch_shapes=[
            pltpu.VMEM((m_per, k), jnp.bfloat16),
            pltpu.VMEM((2, k, NBLK), jnp.float32),
            pltpu.VMEM((N_DEV, m_per, n_per), jnp.int16),
            pltpu.VMEM((N_DEV, m_per, n_per), jnp.int16),
            pltpu.VMEM((N_DEV, 128), jnp.float32),
            pltpu.VMEM((1, 128), jnp.float32),
            pltpu.SemaphoreType.DMA((2, WSPLIT)),
            pltpu.SemaphoreType.DMA((N_DEV - 1,)),
            pltpu.SemaphoreType.DMA((N_DEV,)),
            pltpu.SemaphoreType.DMA((N_DEV - 1,)),
            pltpu.SemaphoreType.DMA((N_DEV,)),
        ],
        compiler_params=pltpu.CompilerParams(
            collective_id=0,
            vmem_limit_bytes=100 * 1024 * 1024,
        ),
    )(x, w_mat)
